# baseline (speedup 1.0000x reference)
import functools

import jax
import jax.numpy as jnp
from jax import lax
from jax.experimental import pallas as pl
from jax.experimental.pallas import tpu as pltpu
from jax.experimental.pallas import tpu_sc as plsc

_DIM = 64
_NC = 2
_NS = 16
_NW = _NC * _NS
_CHUNK = 256
_NBUF = 5


@functools.partial(jax.jit, static_argnames=("total",))
def _sc_gather(x_flat4, table, total):
    n_per_w = total // _NW
    n_chunks = n_per_w // _CHUNK
    assert n_chunks % _NBUF == 0
    mesh = plsc.VectorSubcoreMesh(core_axis_name="c", subcore_axis_name="s")

    @functools.partial(
        pl.kernel,
        mesh=mesh,
        out_type=jax.ShapeDtypeStruct((_NW, n_chunks, _CHUNK, _DIM), jnp.float32),
        scratch_types=[
            pltpu.VMEM((n_chunks, _CHUNK), jnp.int32),
            pltpu.VMEM((_NBUF, _CHUNK, _DIM), jnp.float32),
            pltpu.SemaphoreType.DMA((_NBUF,)),
            pltpu.SemaphoreType.DMA((_NBUF,)),
        ],
        compiler_params=pltpu.CompilerParams(use_tc_tiling_on_sc=False),
    )
    def k(x_hbm, table_hbm, out_hbm, idx_v, rows_v, gsem, osem):
        wid = lax.axis_index("s") * _NC + lax.axis_index("c")
        pltpu.sync_copy(x_hbm.at[wid], idx_v)

        for b in range(_NBUF - 1):
            pltpu.async_copy(table_hbm.at[idx_v.at[b]], rows_v.at[b], gsem.at[b])

        @pl.loop(0, n_chunks, step=_NBUF)
        def _outer(g):
            for b in range(_NBUF):
                j = g + b
                sf = (b - 1) % _NBUF

                @pl.when(j >= 1)
                def _():
                    pltpu.make_async_copy(
                        rows_v.at[sf],
                        out_hbm.at[wid, 0],
                        osem.at[sf],
                    ).wait()

                @pl.when(j + _NBUF - 1 < n_chunks)
                def _():
                    pltpu.async_copy(
                        table_hbm.at[idx_v.at[j + _NBUF - 1]],
                        rows_v.at[sf],
                        gsem.at[sf],
                    )

                pltpu.make_async_copy(
                    table_hbm.at[idx_v.at[j]],
                    rows_v.at[b],
                    gsem.at[b],
                ).wait()
                pltpu.async_copy(
                    rows_v.at[b],
                    out_hbm.at[wid, j],
                    osem.at[b],
                )

        pltpu.make_async_copy(
            rows_v.at[_NBUF - 1],
            out_hbm.at[wid, 0],
            osem.at[_NBUF - 1],
        ).wait()

    return k(x_flat4, table)


def kernel(x, table):
    batch, hist = x.shape
    total = batch * hist
    n_chunks = total // (_NW * _CHUNK)
    xr = x.reshape(_NW, n_chunks, _CHUNK).astype(jnp.int32)
    out = _sc_gather(xr, table, total)
    return out.reshape(batch, hist, _DIM)


# speedup vs baseline: 1.0013x; 1.0013x over previous
import functools

import jax
import jax.numpy as jnp
from jax import lax
from jax.experimental import pallas as pl
from jax.experimental.pallas import tpu as pltpu
from jax.experimental.pallas import tpu_sc as plsc

_DIM = 64
_NC = 2
_NS = 16
_NW = _NC * _NS
_CHUNK = 256
_NBUF = 5


@functools.partial(jax.jit, static_argnames=("total",))
def _sc_gather(x_flat, table, total):
    n_per_w = total // _NW
    n_chunks = n_per_w // _CHUNK
    assert n_chunks % _NBUF == 0
    mesh = plsc.VectorSubcoreMesh(core_axis_name="c", subcore_axis_name="s")

    @functools.partial(
        pl.kernel,
        mesh=mesh,
        out_type=jax.ShapeDtypeStruct((total, _DIM), jnp.float32),
        scratch_types=[
            pltpu.VMEM((n_per_w,), jnp.int32),
            pltpu.VMEM((_NBUF, _CHUNK, _DIM), jnp.float32),
            pltpu.SemaphoreType.DMA((_NBUF,)),
            pltpu.SemaphoreType.DMA((_NBUF,)),
        ],
        compiler_params=pltpu.CompilerParams(use_tc_tiling_on_sc=False),
    )
    def k(x_hbm, table_hbm, out_hbm, idx_v, rows_v, gsem, osem):
        wid = lax.axis_index("s") * _NC + lax.axis_index("c")
        base = wid * n_per_w
        pltpu.sync_copy(x_hbm.at[pl.ds(base, n_per_w)], idx_v)

        for b in range(_NBUF - 1):
            pltpu.async_copy(
                table_hbm.at[idx_v.at[pl.ds(b * _CHUNK, _CHUNK)]],
                rows_v.at[b],
                gsem.at[b],
            )

        @pl.loop(0, n_chunks, step=_NBUF)
        def _outer(g):
            for b in range(_NBUF):
                j = g + b
                sf = (b - 1) % _NBUF

                @pl.when(j >= 1)
                def _():
                    pltpu.make_async_copy(
                        rows_v.at[sf],
                        out_hbm.at[pl.ds(base, _CHUNK)],
                        osem.at[sf],
                    ).wait()

                @pl.when(j + _NBUF - 1 < n_chunks)
                def _():
                    pltpu.async_copy(
                        table_hbm.at[idx_v.at[pl.ds((j + _NBUF - 1) * _CHUNK, _CHUNK)]],
                        rows_v.at[sf],
                        gsem.at[sf],
                    )

                pltpu.make_async_copy(
                    table_hbm.at[idx_v.at[pl.ds(j * _CHUNK, _CHUNK)]],
                    rows_v.at[b],
                    gsem.at[b],
                ).wait()
                pltpu.async_copy(
                    rows_v.at[b],
                    out_hbm.at[pl.ds(base + j * _CHUNK, _CHUNK)],
                    osem.at[b],
                )

        pltpu.make_async_copy(
            rows_v.at[_NBUF - 1],
            out_hbm.at[pl.ds(base, _CHUNK)],
            osem.at[_NBUF - 1],
        ).wait()

    return k(x_flat, table)


def kernel(x, table):
    batch, hist = x.shape
    total = batch * hist
    xf = x.reshape(-1).astype(jnp.int32)
    out = _sc_gather(xf, table, total)
    return out.reshape(batch, hist, _DIM)
